# baseline (speedup 1.0000x reference)
import functools

import jax
import jax.numpy as jnp
from jax import lax
from jax.experimental import pallas as pl
from jax.experimental.pallas import tpu as pltpu
from jax.experimental.pallas import tpu_sc as plsc

N = 10000
E = 320000
D_IN = 128
D_HID = 128
D_OUT = 64

NC = 2
NS = 16
NW = NC * NS
EPT = E // NW
RPT = 632
N_PAD = RPT * NS

RB = 10000


def _make_agg(D, B):
  CH = EPT // B
  mesh = plsc.VectorSubcoreMesh(core_axis_name="c", subcore_axis_name="s")

  @functools.partial(
      pl.kernel,
      out_type=jax.ShapeDtypeStruct((NC, N_PAD, D), jnp.float32),
      mesh=mesh,
      compiler_params=pltpu.CompilerParams(use_tc_tiling_on_sc=False),
      scratch_types=[
          pltpu.VMEM((CH, B), jnp.int32),
          pltpu.VMEM((CH, B), jnp.int32),
          pltpu.VMEM((B, D), jnp.float32),
          pltpu.VMEM((B, D), jnp.float32),
          pltpu.VMEM_SHARED((N_PAD, D), jnp.float32),
          pltpu.SemaphoreType.DMA,
          pltpu.SemaphoreType.DMA,
      ],
  )
  def agg(feat_hbm, src_hbm, dst_hbm, zeros_hbm, out_hbm,
          src_v, dst_v, rows0, rows1, acc_sh, sem0, sem1):
    cid = lax.axis_index("c")
    tid = lax.axis_index("s")
    wid = tid * NC + cid
    pltpu.sync_copy(zeros_hbm, acc_sh.at[pl.ds(tid * RPT, RPT)])
    pltpu.sync_copy(src_hbm.at[wid], src_v)
    pltpu.sync_copy(dst_hbm.at[wid], dst_v)
    plsc.subcore_barrier()

    pltpu.async_copy(feat_hbm.at[src_v.at[0]], rows0, sem0)

    def body(g, carry):
      j0 = 2 * g
      j1 = j0 + 1
      pltpu.async_copy(feat_hbm.at[src_v.at[j1]], rows1, sem1)
      pltpu.make_async_copy(feat_hbm.at[pl.ds(0, B)], rows0, sem0).wait()
      pltpu.sync_copy(rows0, acc_sh.at[dst_v.at[j0]], add=True)
      j2 = jnp.minimum(j1 + 1, CH - 1)
      pltpu.async_copy(feat_hbm.at[src_v.at[j2]], rows0, sem0)
      pltpu.make_async_copy(feat_hbm.at[pl.ds(0, B)], rows1, sem1).wait()
      pltpu.sync_copy(rows1, acc_sh.at[dst_v.at[j1]], add=True)
      return carry

    lax.fori_loop(0, CH // 2, body, 0, unroll=False)
    pltpu.make_async_copy(feat_hbm.at[pl.ds(0, B)], rows0, sem0).wait()
    plsc.subcore_barrier()
    pltpu.sync_copy(acc_sh.at[pl.ds(tid * RPT, RPT)],
                    out_hbm.at[cid, pl.ds(tid * RPT, RPT)])

  return agg


def _make_agg_dbi(D, B):
  CH = EPT // B
  mesh = plsc.VectorSubcoreMesh(core_axis_name="c", subcore_axis_name="s")

  @functools.partial(
      pl.kernel,
      out_type=jax.ShapeDtypeStruct((NC, N_PAD, D), jnp.float32),
      mesh=mesh,
      compiler_params=pltpu.CompilerParams(use_tc_tiling_on_sc=False),
      scratch_types=[
          pltpu.VMEM((2, B), jnp.int32),
          pltpu.VMEM((2, B), jnp.int32),
          pltpu.VMEM((B, D), jnp.float32),
          pltpu.VMEM((B, D), jnp.float32),
          pltpu.VMEM_SHARED((N_PAD, D), jnp.float32),
          pltpu.SemaphoreType.DMA,
          pltpu.SemaphoreType.DMA,
          pltpu.SemaphoreType.DMA,
          pltpu.SemaphoreType.DMA,
          pltpu.SemaphoreType.DMA,
          pltpu.SemaphoreType.DMA,
      ],
  )
  def agg(feat_hbm, src_hbm, dst_hbm, zeros_hbm, out_hbm,
          src2, dst2, rows0, rows1, acc_sh,
          sem0, sem1, isem0, isem1, dsem0, dsem1):
    cid = lax.axis_index("c")
    tid = lax.axis_index("s")
    wid = tid * NC + cid
    pltpu.sync_copy(zeros_hbm, acc_sh.at[pl.ds(tid * RPT, RPT)])
    plsc.subcore_barrier()

    pltpu.sync_copy(src_hbm.at[wid, 0], src2.at[0])
    pltpu.async_copy(feat_hbm.at[src2.at[0]], rows0, sem0)
    pltpu.async_copy(src_hbm.at[wid, 1], src2.at[1], isem1)
    pltpu.async_copy(dst_hbm.at[wid, 0], dst2.at[0], dsem0)
    pltpu.async_copy(dst_hbm.at[wid, 1], dst2.at[1], dsem1)

    def _wait_idx(sem):
      pltpu.make_async_copy(src_hbm.at[0, 0], src2.at[0], sem).wait()

    def body(g, carry):
      j0 = 2 * g
      j1 = j0 + 1
      j2 = jnp.minimum(j0 + 2, CH - 1)
      j3 = jnp.minimum(j1 + 2, CH - 1)
      _wait_idx(isem1)
      pltpu.async_copy(feat_hbm.at[src2.at[1]], rows1, sem1)
      pltpu.make_async_copy(feat_hbm.at[pl.ds(0, B)], rows0, sem0).wait()
      pltpu.async_copy(src_hbm.at[wid, j2], src2.at[0], isem0)
      _wait_idx(dsem0)
      pltpu.sync_copy(rows0, acc_sh.at[dst2.at[0]], add=True)
      pltpu.async_copy(dst_hbm.at[wid, j2], dst2.at[0], dsem0)
      _wait_idx(isem0)
      pltpu.async_copy(feat_hbm.at[src2.at[0]], rows0, sem0)
      pltpu.make_async_copy(feat_hbm.at[pl.ds(0, B)], rows1, sem1).wait()
      pltpu.async_copy(src_hbm.at[wid, j3], src2.at[1], isem1)
      _wait_idx(dsem1)
      pltpu.sync_copy(rows1, acc_sh.at[dst2.at[1]], add=True)
      pltpu.async_copy(dst_hbm.at[wid, j3], dst2.at[1], dsem1)
      return carry

    lax.fori_loop(0, CH // 2, body, 0, unroll=False)
    pltpu.make_async_copy(feat_hbm.at[pl.ds(0, B)], rows0, sem0).wait()
    _wait_idx(isem1)
    _wait_idx(dsem0)
    _wait_idx(dsem1)
    plsc.subcore_barrier()
    pltpu.sync_copy(acc_sh.at[pl.ds(tid * RPT, RPT)],
                    out_hbm.at[cid, pl.ds(tid * RPT, RPT)])

  return agg


D_DEG = 16


def _make_deg(B):
  CH = EPT // B
  mesh = plsc.VectorSubcoreMesh(core_axis_name="c", subcore_axis_name="s")

  @functools.partial(
      pl.kernel,
      out_type=jax.ShapeDtypeStruct((NC, N_PAD, D_DEG), jnp.float32),
      mesh=mesh,
      compiler_params=pltpu.CompilerParams(use_tc_tiling_on_sc=False),
      scratch_types=[
          pltpu.VMEM((CH, B), jnp.int32),
          pltpu.VMEM((B, D_DEG), jnp.float32),
          pltpu.VMEM_SHARED((N_PAD, D_DEG), jnp.float32),
      ],
  )
  def deg(ones_hbm, dst_hbm, zeros_hbm, out_hbm, dst_v, ones_v, acc_sh):
    cid = lax.axis_index("c")
    tid = lax.axis_index("s")
    wid = tid * NC + cid
    pltpu.sync_copy(zeros_hbm, acc_sh.at[pl.ds(tid * RPT, RPT)])
    pltpu.sync_copy(dst_hbm.at[wid], dst_v)
    pltpu.sync_copy(ones_hbm, ones_v)
    plsc.subcore_barrier()

    def body(j, carry):
      pltpu.sync_copy(ones_v, acc_sh.at[dst_v.at[j]], add=True)
      return carry

    lax.fori_loop(0, CH, body, 0, unroll=False)
    plsc.subcore_barrier()
    pltpu.sync_copy(acc_sh.at[pl.ds(tid * RPT, RPT)],
                    out_hbm.at[cid, pl.ds(tid * RPT, RPT)])

  return deg


B128 = 125
B64 = 125
BDEG = 125

_agg_deg = _make_deg(BDEG)
_agg128 = _make_agg_dbi(D_HID, B128)
_agg64 = _make_agg(D_OUT, B64)


def _l1_body(d0, d1, x, w1, hp, dis):
  deg = d0[...] + d1[...] + 1.0
  s = lax.rsqrt(jnp.max(deg, axis=1, keepdims=True))
  dis[...] = s
  h = jnp.dot(x[...], w1[...], preferred_element_type=jnp.float32)
  hp[...] = s * h


def _l1(x, w1, d0, d1):
  return pl.pallas_call(
      _l1_body,
      grid=(N // RB,),
      in_specs=[
          pl.BlockSpec((RB, D_DEG), lambda i: (i, 0)),
          pl.BlockSpec((RB, D_DEG), lambda i: (i, 0)),
          pl.BlockSpec((RB, D_IN), lambda i: (i, 0)),
          pl.BlockSpec((D_IN, D_HID), lambda i: (0, 0)),
      ],
      out_specs=[
          pl.BlockSpec((RB, D_HID), lambda i: (i, 0)),
          pl.BlockSpec((RB, 1), lambda i: (i, 0)),
      ],
      out_shape=[
          jax.ShapeDtypeStruct((N, D_HID), jnp.float32),
          jax.ShapeDtypeStruct((N, 1), jnp.float32),
      ],
  )(d0, d1, x, w1)


def _l2_body(a0, a1, hp1, dis, b1, w2, hp2):
  s = dis[...]
  z = jnp.maximum(s * (a0[...] + a1[...] + hp1[...]) + b1[...], 0.0)
  hp2[...] = s * jnp.dot(z, w2[...], preferred_element_type=jnp.float32)


def _l2(a0, a1, hp1, dis, b1, w2):
  return pl.pallas_call(
      _l2_body,
      grid=(N // RB,),
      in_specs=[
          pl.BlockSpec((RB, D_HID), lambda i: (i, 0)),
          pl.BlockSpec((RB, D_HID), lambda i: (i, 0)),
          pl.BlockSpec((RB, D_HID), lambda i: (i, 0)),
          pl.BlockSpec((RB, 1), lambda i: (i, 0)),
          pl.BlockSpec((1, D_HID), lambda i: (0, 0)),
          pl.BlockSpec((D_HID, D_OUT), lambda i: (0, 0)),
      ],
      out_specs=pl.BlockSpec((RB, D_OUT), lambda i: (i, 0)),
      out_shape=jax.ShapeDtypeStruct((N, D_OUT), jnp.float32),
  )(a0, a1, hp1, dis, b1, w2)


def _fin_body(a0, a1, hp2, dis, b2, o):
  o[...] = dis[...] * (a0[...] + a1[...] + hp2[...]) + b2[...]


def _fin(a0, a1, hp2, dis, b2):
  return pl.pallas_call(
      _fin_body,
      grid=(N // RB,),
      in_specs=[
          pl.BlockSpec((RB, D_OUT), lambda i: (i, 0)),
          pl.BlockSpec((RB, D_OUT), lambda i: (i, 0)),
          pl.BlockSpec((RB, D_OUT), lambda i: (i, 0)),
          pl.BlockSpec((RB, 1), lambda i: (i, 0)),
          pl.BlockSpec((1, D_OUT), lambda i: (0, 0)),
      ],
      out_specs=pl.BlockSpec((RB, D_OUT), lambda i: (i, 0)),
      out_shape=jax.ShapeDtypeStruct((N, D_OUT), jnp.float32),
  )(a0, a1, hp2, dis, b2)


def kernel(x, edge_index, W1, b1, W2, b2):
  src_a = edge_index[0].reshape(NW, EPT // B128, B128)
  dst_a = edge_index[1].reshape(NW, EPT // B128, B128)
  src_b = edge_index[0].reshape(NW, EPT // B64, B64)
  dst_b = edge_index[1].reshape(NW, EPT // B64, B64)
  dst_d = edge_index[1].reshape(NW, EPT // BDEG, BDEG)
  ones = jnp.ones((BDEG, D_DEG), jnp.float32)
  z1 = jnp.zeros((RPT, D_DEG), jnp.float32)
  z128 = jnp.zeros((RPT, D_HID), jnp.float32)
  z64 = jnp.zeros((RPT, D_OUT), jnp.float32)

  degp = _agg_deg(ones, dst_d, z1)
  hp1, dis = _l1(x, W1, degp[0, :N], degp[1, :N])
  a1 = _agg128(hp1, src_a, dst_a, z128)
  hp2 = _l2(a1[0, :N], a1[1, :N], hp1, dis, b1.reshape(1, -1), W2)
  a2 = _agg64(hp2, src_b, dst_b, z64)
  return _fin(a2[0, :N], a2[1, :N], hp2, dis, b2.reshape(1, -1))


# speedup vs baseline: 1.0092x; 1.0092x over previous
import functools

import jax
import jax.numpy as jnp
from jax import lax
from jax.experimental import pallas as pl
from jax.experimental.pallas import tpu as pltpu
from jax.experimental.pallas import tpu_sc as plsc

N = 10000
E = 320000
D_IN = 128
D_HID = 128
D_OUT = 64

NC = 2
NS = 16
NW = NC * NS
EPT = E // NW
RPT = 632
N_PAD = RPT * NS

RB = 5000


def _make_agg(D, B):
  CH = EPT // B
  mesh = plsc.VectorSubcoreMesh(core_axis_name="c", subcore_axis_name="s")

  @functools.partial(
      pl.kernel,
      out_type=jax.ShapeDtypeStruct((NC, N_PAD, D), jnp.float32),
      mesh=mesh,
      compiler_params=pltpu.CompilerParams(use_tc_tiling_on_sc=False),
      scratch_types=[
          pltpu.VMEM((CH, B), jnp.int32),
          pltpu.VMEM((CH, B), jnp.int32),
          pltpu.VMEM((B, D), jnp.float32),
          pltpu.VMEM((B, D), jnp.float32),
          pltpu.VMEM_SHARED((N_PAD, D), jnp.float32),
          pltpu.SemaphoreType.DMA,
          pltpu.SemaphoreType.DMA,
      ],
  )
  def agg(feat_hbm, src_hbm, dst_hbm, zeros_hbm, out_hbm,
          src_v, dst_v, rows0, rows1, acc_sh, sem0, sem1):
    cid = lax.axis_index("c")
    tid = lax.axis_index("s")
    wid = tid * NC + cid
    pltpu.sync_copy(zeros_hbm, acc_sh.at[pl.ds(tid * RPT, RPT)])
    pltpu.sync_copy(src_hbm.at[wid], src_v)
    pltpu.sync_copy(dst_hbm.at[wid], dst_v)
    plsc.subcore_barrier()

    pltpu.async_copy(feat_hbm.at[src_v.at[0]], rows0, sem0)

    def body(g, carry):
      j0 = 2 * g
      j1 = j0 + 1
      pltpu.async_copy(feat_hbm.at[src_v.at[j1]], rows1, sem1)
      pltpu.make_async_copy(feat_hbm.at[pl.ds(0, B)], rows0, sem0).wait()
      pltpu.sync_copy(rows0, acc_sh.at[dst_v.at[j0]], add=True)
      j2 = jnp.minimum(j1 + 1, CH - 1)
      pltpu.async_copy(feat_hbm.at[src_v.at[j2]], rows0, sem0)
      pltpu.make_async_copy(feat_hbm.at[pl.ds(0, B)], rows1, sem1).wait()
      pltpu.sync_copy(rows1, acc_sh.at[dst_v.at[j1]], add=True)
      return carry

    lax.fori_loop(0, CH // 2, body, 0, unroll=False)
    pltpu.make_async_copy(feat_hbm.at[pl.ds(0, B)], rows0, sem0).wait()
    plsc.subcore_barrier()
    pltpu.sync_copy(acc_sh.at[pl.ds(tid * RPT, RPT)],
                    out_hbm.at[cid, pl.ds(tid * RPT, RPT)])

  return agg


def _make_agg_dbi(D, B):
  CH = EPT // B
  mesh = plsc.VectorSubcoreMesh(core_axis_name="c", subcore_axis_name="s")

  @functools.partial(
      pl.kernel,
      out_type=jax.ShapeDtypeStruct((NC, N_PAD, D), jnp.float32),
      mesh=mesh,
      compiler_params=pltpu.CompilerParams(use_tc_tiling_on_sc=False),
      scratch_types=[
          pltpu.VMEM((2, B), jnp.int32),
          pltpu.VMEM((2, B), jnp.int32),
          pltpu.VMEM((B, D), jnp.float32),
          pltpu.VMEM((B, D), jnp.float32),
          pltpu.VMEM_SHARED((N_PAD, D), jnp.float32),
          pltpu.SemaphoreType.DMA,
          pltpu.SemaphoreType.DMA,
          pltpu.SemaphoreType.DMA,
          pltpu.SemaphoreType.DMA,
          pltpu.SemaphoreType.DMA,
          pltpu.SemaphoreType.DMA,
      ],
  )
  def agg(feat_hbm, src_hbm, dst_hbm, zeros_hbm, out_hbm,
          src2, dst2, rows0, rows1, acc_sh,
          sem0, sem1, isem0, isem1, dsem0, dsem1):
    cid = lax.axis_index("c")
    tid = lax.axis_index("s")
    wid = tid * NC + cid
    pltpu.sync_copy(zeros_hbm, acc_sh.at[pl.ds(tid * RPT, RPT)])
    plsc.subcore_barrier()

    pltpu.sync_copy(src_hbm.at[wid, 0], src2.at[0])
    pltpu.async_copy(feat_hbm.at[src2.at[0]], rows0, sem0)
    pltpu.async_copy(src_hbm.at[wid, 1], src2.at[1], isem1)
    pltpu.async_copy(dst_hbm.at[wid, 0], dst2.at[0], dsem0)
    pltpu.async_copy(dst_hbm.at[wid, 1], dst2.at[1], dsem1)

    def _wait_idx(sem):
      pltpu.make_async_copy(src_hbm.at[0, 0], src2.at[0], sem).wait()

    def body(g, carry):
      j0 = 2 * g
      j1 = j0 + 1
      j2 = jnp.minimum(j0 + 2, CH - 1)
      j3 = jnp.minimum(j1 + 2, CH - 1)
      _wait_idx(isem1)
      pltpu.async_copy(feat_hbm.at[src2.at[1]], rows1, sem1)
      pltpu.make_async_copy(feat_hbm.at[pl.ds(0, B)], rows0, sem0).wait()
      pltpu.async_copy(src_hbm.at[wid, j2], src2.at[0], isem0)
      _wait_idx(dsem0)
      pltpu.sync_copy(rows0, acc_sh.at[dst2.at[0]], add=True)
      pltpu.async_copy(dst_hbm.at[wid, j2], dst2.at[0], dsem0)
      _wait_idx(isem0)
      pltpu.async_copy(feat_hbm.at[src2.at[0]], rows0, sem0)
      pltpu.make_async_copy(feat_hbm.at[pl.ds(0, B)], rows1, sem1).wait()
      pltpu.async_copy(src_hbm.at[wid, j3], src2.at[1], isem1)
      _wait_idx(dsem1)
      pltpu.sync_copy(rows1, acc_sh.at[dst2.at[1]], add=True)
      pltpu.async_copy(dst_hbm.at[wid, j3], dst2.at[1], dsem1)
      return carry

    lax.fori_loop(0, CH // 2, body, 0, unroll=False)
    pltpu.make_async_copy(feat_hbm.at[pl.ds(0, B)], rows0, sem0).wait()
    _wait_idx(isem1)
    _wait_idx(dsem0)
    _wait_idx(dsem1)
    plsc.subcore_barrier()
    pltpu.sync_copy(acc_sh.at[pl.ds(tid * RPT, RPT)],
                    out_hbm.at[cid, pl.ds(tid * RPT, RPT)])

  return agg


D_DEG = 16


def _make_deg(B):
  CH = EPT // B
  mesh = plsc.VectorSubcoreMesh(core_axis_name="c", subcore_axis_name="s")

  @functools.partial(
      pl.kernel,
      out_type=jax.ShapeDtypeStruct((NC, N_PAD, D_DEG), jnp.float32),
      mesh=mesh,
      compiler_params=pltpu.CompilerParams(use_tc_tiling_on_sc=False),
      scratch_types=[
          pltpu.VMEM((CH, B), jnp.int32),
          pltpu.VMEM((B, D_DEG), jnp.float32),
          pltpu.VMEM_SHARED((N_PAD, D_DEG), jnp.float32),
      ],
  )
  def deg(ones_hbm, dst_hbm, zeros_hbm, out_hbm, dst_v, ones_v, acc_sh):
    cid = lax.axis_index("c")
    tid = lax.axis_index("s")
    wid = tid * NC + cid
    pltpu.sync_copy(zeros_hbm, acc_sh.at[pl.ds(tid * RPT, RPT)])
    pltpu.sync_copy(dst_hbm.at[wid], dst_v)
    pltpu.sync_copy(ones_hbm, ones_v)
    plsc.subcore_barrier()

    def body(j, carry):
      pltpu.sync_copy(ones_v, acc_sh.at[dst_v.at[j]], add=True)
      return carry

    lax.fori_loop(0, CH, body, 0, unroll=False)
    plsc.subcore_barrier()
    pltpu.sync_copy(acc_sh.at[pl.ds(tid * RPT, RPT)],
                    out_hbm.at[cid, pl.ds(tid * RPT, RPT)])

  return deg


B128 = 125
B64 = 125
BDEG = 125

_agg_deg = _make_deg(BDEG)
_agg128 = _make_agg_dbi(D_HID, B128)
_agg64 = _make_agg(D_OUT, B64)


def _l1_body(d0, d1, x, w1, hp, dis):
  deg = d0[...] + d1[...] + 1.0
  s = lax.rsqrt(jnp.max(deg, axis=1, keepdims=True))
  dis[...] = s
  h = jnp.dot(x[...], w1[...], preferred_element_type=jnp.float32)
  hp[...] = s * h


def _l1(x, w1, d0, d1):
  return pl.pallas_call(
      _l1_body,
      grid=(N // RB,),
      in_specs=[
          pl.BlockSpec((RB, D_DEG), lambda i: (i, 0)),
          pl.BlockSpec((RB, D_DEG), lambda i: (i, 0)),
          pl.BlockSpec((RB, D_IN), lambda i: (i, 0)),
          pl.BlockSpec((D_IN, D_HID), lambda i: (0, 0)),
      ],
      out_specs=[
          pl.BlockSpec((RB, D_HID), lambda i: (i, 0)),
          pl.BlockSpec((RB, 1), lambda i: (i, 0)),
      ],
      out_shape=[
          jax.ShapeDtypeStruct((N, D_HID), jnp.float32),
          jax.ShapeDtypeStruct((N, 1), jnp.float32),
      ],
  )(d0, d1, x, w1)


def _l2_body(a0, a1, hp1, dis, b1, w2, hp2):
  s = dis[...]
  z = jnp.maximum(s * (a0[...] + a1[...] + hp1[...]) + b1[...], 0.0)
  hp2[...] = s * jnp.dot(z, w2[...], preferred_element_type=jnp.float32)


def _l2(a0, a1, hp1, dis, b1, w2):
  return pl.pallas_call(
      _l2_body,
      grid=(N // RB,),
      in_specs=[
          pl.BlockSpec((RB, D_HID), lambda i: (i, 0)),
          pl.BlockSpec((RB, D_HID), lambda i: (i, 0)),
          pl.BlockSpec((RB, D_HID), lambda i: (i, 0)),
          pl.BlockSpec((RB, 1), lambda i: (i, 0)),
          pl.BlockSpec((1, D_HID), lambda i: (0, 0)),
          pl.BlockSpec((D_HID, D_OUT), lambda i: (0, 0)),
      ],
      out_specs=pl.BlockSpec((RB, D_OUT), lambda i: (i, 0)),
      out_shape=jax.ShapeDtypeStruct((N, D_OUT), jnp.float32),
  )(a0, a1, hp1, dis, b1, w2)


def _fin_body(a0, a1, hp2, dis, b2, o):
  o[...] = dis[...] * (a0[...] + a1[...] + hp2[...]) + b2[...]


def _fin(a0, a1, hp2, dis, b2):
  return pl.pallas_call(
      _fin_body,
      grid=(N // RB,),
      in_specs=[
          pl.BlockSpec((RB, D_OUT), lambda i: (i, 0)),
          pl.BlockSpec((RB, D_OUT), lambda i: (i, 0)),
          pl.BlockSpec((RB, D_OUT), lambda i: (i, 0)),
          pl.BlockSpec((RB, 1), lambda i: (i, 0)),
          pl.BlockSpec((1, D_OUT), lambda i: (0, 0)),
      ],
      out_specs=pl.BlockSpec((RB, D_OUT), lambda i: (i, 0)),
      out_shape=jax.ShapeDtypeStruct((N, D_OUT), jnp.float32),
  )(a0, a1, hp2, dis, b2)


def kernel(x, edge_index, W1, b1, W2, b2):
  src_a = edge_index[0].reshape(NW, EPT // B128, B128)
  dst_a = edge_index[1].reshape(NW, EPT // B128, B128)
  src_b = edge_index[0].reshape(NW, EPT // B64, B64)
  dst_b = edge_index[1].reshape(NW, EPT // B64, B64)
  dst_d = edge_index[1].reshape(NW, EPT // BDEG, BDEG)
  ones = jnp.ones((BDEG, D_DEG), jnp.float32)
  z1 = jnp.zeros((RPT, D_DEG), jnp.float32)
  z128 = jnp.zeros((RPT, D_HID), jnp.float32)
  z64 = jnp.zeros((RPT, D_OUT), jnp.float32)

  degp = _agg_deg(ones, dst_d, z1)
  hp1, dis = _l1(x, W1, degp[0, :N], degp[1, :N])
  a1 = _agg128(hp1, src_a, dst_a, z128)
  hp2 = _l2(a1[0, :N], a1[1, :N], hp1, dis, b1.reshape(1, -1), W2)
  a2 = _agg64(hp2, src_b, dst_b, z64)
  return _fin(a2[0, :N], a2[1, :N], hp2, dis, b2.reshape(1, -1))
